# baseline (speedup 1.0000x reference)
import functools

import jax
import jax.numpy as jnp
from jax import lax
from jax.experimental import pallas as pl
from jax.experimental.pallas import tpu as pltpu
from jax.experimental.pallas import tpu_sc as plsc

_NC = 2
_NS = 16
_NW = _NC * _NS
_CH = 128
_L = 16


@functools.lru_cache(maxsize=None)
def _make_gather(B, V, D):
    bpw = B // _NW
    nchunk = bpw // _CH
    nd_blk = D // 8
    blk = 8 * bpw
    mesh = plsc.VectorSubcoreMesh(core_axis_name="c", subcore_axis_name="s")

    @functools.partial(
        pl.kernel,
        mesh=mesh,
        out_type=jax.ShapeDtypeStruct((B * D,), jnp.float32),
        scratch_types=[
            pltpu.VMEM((nchunk, _CH), jnp.int32),
            pltpu.VMEM((nchunk, _CH, D), jnp.float32),
            pltpu.VMEM((bpw * D,), jnp.float32),
            pltpu.SemaphoreType.DMA,
            pltpu.SemaphoreType.DMA,
        ],
        compiler_params=pltpu.CompilerParams(
            use_tc_tiling_on_sc=False,
            needs_layout_passes=False,
            skip_device_barrier=True,
            disable_semaphore_checks=True,
        ),
    )
    def gather(idx_hbm, table_hbm, out_hbm, idx_v, rows_v, phys_v, gsem, osem):
        wid = lax.axis_index("s") * _NC + lax.axis_index("c")
        base = wid * bpw
        pltpu.sync_copy(idx_hbm.at[wid], idx_v)
        gathers = [
            pltpu.async_copy(table_hbm.at[idx_v.at[j]], rows_v.at[j], gsem)
            for j in range(nchunk)
        ]
        for g in gathers:
            g.wait()

        lanes = lax.iota(jnp.int32, _L)

        def permute(t, _):
            d_in = t % 8
            b_blk = (t // 8) % nchunk
            d_blk = t // (8 * nchunk)
            d = lanes * 0 + (d_blk * 8 + d_in)
            jvec = lanes * 0 + b_blk
            for h in range(_CH // _L):
                vals = plsc.load_gather(rows_v, [jvec, h * _L + lanes, d])
                phys_v[pl.ds(t * _CH + h * _L, _L)] = vals
            return 0

        lax.fori_loop(0, nd_blk * nchunk * 8, permute, 0)

        stores = [
            pltpu.async_copy(
                phys_v.at[pl.ds(d_blk * blk, blk)],
                out_hbm.at[
                    pl.ds((d_blk * (B // _CH) + wid * nchunk) * 1024, blk)
                ],
                osem,
            )
            for d_blk in range(nd_blk)
        ]
        for s in stores:
            s.wait()

    return gather


@functools.lru_cache(maxsize=None)
def _make_sc_relayout(V, D, cw=1024):
    ntail = V % 128
    vmain = V - ntail
    nwin_total = (vmain + cw - 1) // cw
    nwin = (nwin_total + _NW - 1) // _NW
    last_w = nwin_total - 1
    last_cols = vmain - last_w * cw
    mesh = plsc.VectorSubcoreMesh(core_axis_name="c", subcore_axis_name="s")

    rpw = cw * D // 128
    rlast = last_cols * D // 128
    rtail = ntail * D // 128
    mesh2 = mesh

    @functools.partial(
        pl.kernel,
        mesh=mesh2,
        out_type=jax.ShapeDtypeStruct((V * D // 128, 128), jnp.float32),
        scratch_types=[
            pltpu.VMEM((D, cw), jnp.float32),
            pltpu.VMEM((rpw, 128), jnp.float32),
            pltpu.VMEM((rtail, 128), jnp.float32),
        ],
        compiler_params=pltpu.CompilerParams(needs_layout_passes=False),
    )
    def relayout(tableT_hbm, tail_hbm, out_hbm, win_v, rows_v, tail_v):
        wid = lax.axis_index("s") * _NC + lax.axis_index("c")
        lanes = lax.iota(jnp.int32, _L)

        @pl.when(wid == 0)
        def _():
            pltpu.sync_copy(tail_hbm, tail_v)
            pltpu.sync_copy(
                tail_v, out_hbm.at[pl.ds(vmain * D // 128, rtail), :]
            )

        def do_rows(g, _):
            for u in range(8):
                r = g * 8 + u
                rvec = lanes * 0 + r
                lo = plsc.load_gather(win_v, [lanes, rvec])
                hi = plsc.load_gather(win_v, [lanes + _L, rvec])
                row = g * (8 * D // 128) + (u * D) // 128
                col = (u * D) % 128
                rows_v[row, pl.ds(col, _L)] = lo
                rows_v[row, pl.ds(col + _L, _L)] = hi
            return 0

        def do_window(c, _):
            w = wid * nwin + c
            col0 = pl.multiple_of(w * cw, 128)

            @pl.when(w < last_w)
            def _():
                pltpu.sync_copy(tableT_hbm.at[:, pl.ds(col0, cw)], win_v)
                lax.fori_loop(0, cw // 8, do_rows, 0)
                pltpu.sync_copy(rows_v, out_hbm.at[pl.ds(w * rpw, rpw), :])

            @pl.when(w == last_w)
            def _():
                pltpu.sync_copy(
                    tableT_hbm.at[:, pl.ds(col0, last_cols)],
                    win_v.at[:, pl.ds(0, last_cols)],
                )
                lax.fori_loop(0, last_cols // 8, do_rows, 0)
                pltpu.sync_copy(
                    rows_v.at[pl.ds(0, rlast)],
                    out_hbm.at[pl.ds(w * rpw, rlast), :],
                )
            return 0

        lax.fori_loop(0, nwin, do_window, 0)

    return relayout


def kernel(step, table):
    B = step.shape[0]
    V, D = table.shape
    idx = step.reshape(_NW, B // (_NW * _CH), _CH).astype(jnp.int32)
    ntail = V % 128
    tail = table[V - ntail:].reshape(ntail * D // 128, 128)
    table_rm = _make_sc_relayout(V, D)(table.T, tail).reshape(V, D)
    flat_out = _make_gather(B, V, D)(idx, table_rm)
    return (
        flat_out.reshape(D // 8, B // 128, 8, 128)
        .transpose(1, 3, 0, 2)
        .reshape(B, D)
    )


# speedup vs baseline: 1.5667x; 1.5667x over previous
import functools

import jax
import jax.numpy as jnp
from jax import lax
from jax.experimental import pallas as pl
from jax.experimental.pallas import tpu as pltpu
from jax.experimental.pallas import tpu_sc as plsc

_NC = 2
_NS = 16
_NW = _NC * _NS
_CH = 128
_L = 16


@functools.lru_cache(maxsize=None)
def _make_gather(B, V, D):
    bpw = B // _NW
    nchunk = bpw // _CH
    nd_blk = D // 8
    blk = 8 * bpw
    mesh = plsc.VectorSubcoreMesh(core_axis_name="c", subcore_axis_name="s")

    @functools.partial(
        pl.kernel,
        mesh=mesh,
        out_type=jax.ShapeDtypeStruct((B * D,), jnp.float32),
        scratch_types=[
            pltpu.VMEM((nchunk, _CH), jnp.int32),
            pltpu.VMEM((nchunk, _CH, D), jnp.float32),
            pltpu.VMEM((bpw * D,), jnp.float32),
            pltpu.SemaphoreType.DMA,
            pltpu.SemaphoreType.DMA,
        ],
        compiler_params=pltpu.CompilerParams(
            use_tc_tiling_on_sc=False,
            needs_layout_passes=False,
            skip_device_barrier=True,
            disable_semaphore_checks=True,
        ),
    )
    def gather(idx_hbm, table_hbm, out_hbm, idx_v, rows_v, phys_v, gsem, osem):
        wid = lax.axis_index("s") * _NC + lax.axis_index("c")
        base = wid * bpw
        pltpu.sync_copy(idx_hbm.at[wid], idx_v)
        gathers = [
            pltpu.async_copy(table_hbm.at[idx_v.at[j]], rows_v.at[j], gsem)
            for j in range(nchunk)
        ]
        for g in gathers:
            g.wait()

        lanes = lax.iota(jnp.int32, _L)

        def permute(t, _):
            d_in = t % 8
            b_blk = (t // 8) % nchunk
            d_blk = t // (8 * nchunk)
            d = lanes * 0 + (d_blk * 8 + d_in)
            jvec = lanes * 0 + b_blk
            for h in range(_CH // _L):
                vals = plsc.load_gather(rows_v, [jvec, h * _L + lanes, d])
                phys_v[pl.ds(t * _CH + h * _L, _L)] = vals
            return 0

        lax.fori_loop(0, nd_blk * nchunk * 8, permute, 0)

        stores = [
            pltpu.async_copy(
                phys_v.at[pl.ds(d_blk * blk, blk)],
                out_hbm.at[
                    pl.ds((d_blk * (B // _CH) + wid * nchunk) * 1024, blk)
                ],
                osem,
            )
            for d_blk in range(nd_blk)
        ]
        for s in stores:
            s.wait()

    return gather


def kernel(step, table):
    B = step.shape[0]
    V, D = table.shape
    idx = step.reshape(_NW, B // (_NW * _CH), _CH).astype(jnp.int32)
    flat_out = _make_gather(B, V, D)(idx, table)
    return (
        flat_out.reshape(D // 8, B // 128, 8, 128)
        .transpose(1, 3, 0, 2)
        .reshape(B, D)
    )


# speedup vs baseline: 1.5699x; 1.0021x over previous
import functools

import jax
import jax.numpy as jnp
from jax import lax
from jax.experimental import pallas as pl
from jax.experimental.pallas import tpu as pltpu
from jax.experimental.pallas import tpu_sc as plsc

_NC = 2
_NS = 16
_NW = _NC * _NS
_CH = 128
_L = 16


@functools.lru_cache(maxsize=None)
def _make_gather(B, V, D):
    bpw = B // _NW
    nchunk = bpw // _CH
    nd_blk = D // 8
    blk = 8 * bpw
    mesh = plsc.VectorSubcoreMesh(core_axis_name="c", subcore_axis_name="s")

    @functools.partial(
        pl.kernel,
        mesh=mesh,
        out_type=jax.ShapeDtypeStruct((B * D,), jnp.float32),
        scratch_types=[
            pltpu.VMEM((nchunk, _CH), jnp.int32),
            pltpu.VMEM((nchunk, _CH, D), jnp.float32),
            pltpu.VMEM((bpw * D,), jnp.float32),
            pltpu.SemaphoreType.DMA,
            pltpu.SemaphoreType.DMA,
        ],
        compiler_params=pltpu.CompilerParams(
            use_tc_tiling_on_sc=False,
            needs_layout_passes=False,
            skip_device_barrier=True,
            disable_semaphore_checks=True,
            allow_input_fusion=[False, True],
        ),
    )
    def gather(idx_hbm, table_hbm, out_hbm, idx_v, rows_v, phys_v, gsem, osem):
        wid = lax.axis_index("s") * _NC + lax.axis_index("c")
        base = wid * bpw
        pltpu.sync_copy(idx_hbm.at[wid], idx_v)
        gathers = [
            pltpu.async_copy(table_hbm.at[idx_v.at[j]], rows_v.at[j], gsem)
            for j in range(nchunk)
        ]
        for g in gathers:
            g.wait()

        lanes = lax.iota(jnp.int32, _L)

        def permute(t, _):
            d_in = t % 8
            b_blk = (t // 8) % nchunk
            d_blk = t // (8 * nchunk)
            d = lanes * 0 + (d_blk * 8 + d_in)
            jvec = lanes * 0 + b_blk
            for h in range(_CH // _L):
                vals = plsc.load_gather(rows_v, [jvec, h * _L + lanes, d])
                phys_v[pl.ds(t * _CH + h * _L, _L)] = vals
            return 0

        lax.fori_loop(0, nd_blk * nchunk * 8, permute, 0)

        stores = [
            pltpu.async_copy(
                phys_v.at[pl.ds(d_blk * blk, blk)],
                out_hbm.at[
                    pl.ds((d_blk * (B // _CH) + wid * nchunk) * 1024, blk)
                ],
                osem,
            )
            for d_blk in range(nd_blk)
        ]
        for s in stores:
            s.wait()

    return gather


def kernel(step, table):
    B = step.shape[0]
    V, D = table.shape
    idx = step.reshape(_NW, B // (_NW * _CH), _CH).astype(jnp.int32)
    flat_out = _make_gather(B, V, D)(idx, table)
    return (
        flat_out.reshape(D // 8, B // 128, 8, 128)
        .transpose(1, 3, 0, 2)
        .reshape(B, D)
    )
